# baseline (speedup 1.0000x reference)
import functools

import jax
from jax import lax
import jax.numpy as jnp
from jax.experimental import pallas as pl
from jax.experimental.pallas import tpu as pltpu
from jax.experimental.pallas import tpu_sc as plsc

NUM_EMB = 1024
EMB_DIM = 64
BPB = 2
HW = 1024

NC = 2
NS = 16
NW = NC * NS
D_PER_W = EMB_DIM // 2


def _vq_codes_kernel(x_ref, cb_ref, s2_ref, iota_row_ref,
                     codes_ref, loss_ref):
    x_blk = x_ref[...]
    cb = cb_ref[...]
    flat = jnp.transpose(x_blk, (0, 2, 1)).reshape(BPB * HW, EMB_DIM)

    m2 = jax.lax.dot_general(
        flat, cb + cb,
        dimension_numbers=(((1,), (1,)), ((), ())),
        preferred_element_type=jnp.float32,
    )
    s1 = jnp.sum(flat * flat, axis=1, keepdims=True)
    s2 = s2_ref[...]
    d2 = (s1 - m2) + s2

    minv = jnp.min(d2, axis=1, keepdims=True)
    iota_row = iota_row_ref[...]
    codes_f = jnp.min(jnp.where(d2 == minv, iota_row, jnp.float32(NUM_EMB)),
                      axis=1)
    codes_ref[0, 0, :] = codes_f.astype(jnp.int32)

    loss_ref[0, 0, 0] = jnp.sum(minv)


def _sc_gather_kernel(cbt_hbm, codes_hbm, out_hbm, cb_v, codes_v, out_v):
    wid = lax.axis_index("s") * NC + lax.axis_index("c")
    b = wid // 2
    half = wid % 2
    pltpu.sync_copy(codes_hbm.at[b], codes_v)
    pltpu.sync_copy(
        cbt_hbm.at[pl.ds(half * D_PER_W * NUM_EMB, D_PER_W * NUM_EMB)], cb_v)

    def body(j, carry):
        base = jnp.full((16,), j * NUM_EMB, jnp.int32)
        for k in range(HW // 16):
            idx = base + codes_v[pl.ds(k * 16, 16)]
            out_v[pl.ds(j * HW + k * 16, 16)] = plsc.load_gather(cb_v, [idx])
        return carry

    lax.fori_loop(0, D_PER_W, body, 0)
    r0 = (b * EMB_DIM + half * D_PER_W) * HW
    pltpu.sync_copy(out_v, out_hbm.at[pl.ds(r0, D_PER_W * HW)])


@functools.partial(jax.jit, static_argnames=())
def kernel(x, codebook):
    B, D, H, W = x.shape
    hw = H * W
    nsteps = B // BPB
    x3 = x.reshape(B, D, hw)
    s2 = jnp.sum(codebook ** 2, axis=1)[None, :]
    iota_row = jax.lax.iota(jnp.float32, NUM_EMB)[None, :]

    codes3, loss_sum = pl.pallas_call(
        _vq_codes_kernel,
        grid=(nsteps,),
        in_specs=[
            pl.BlockSpec((BPB, D, hw), lambda b: (b, 0, 0)),
            pl.BlockSpec((NUM_EMB, EMB_DIM), lambda b: (0, 0)),
            pl.BlockSpec((1, NUM_EMB), lambda b: (0, 0)),
            pl.BlockSpec((1, NUM_EMB), lambda b: (0, 0)),
        ],
        out_specs=[
            pl.BlockSpec((1, 1, BPB * hw), lambda b: (b, 0, 0)),
            pl.BlockSpec((1, 1, 1), lambda b: (b, 0, 0), memory_space=pltpu.SMEM),
        ],
        out_shape=[
            jax.ShapeDtypeStruct((nsteps, 1, BPB * hw), jnp.int32),
            jax.ShapeDtypeStruct((nsteps, 1, 1), jnp.float32),
        ],
        compiler_params=pltpu.CompilerParams(
            dimension_semantics=("parallel",),
        ),
    )(x3, codebook, s2, iota_row)

    codes2d = codes3.reshape(B, hw)
    cbt = codebook.T.reshape(EMB_DIM * NUM_EMB)

    sc_gather = functools.partial(
        pl.kernel,
        mesh=plsc.VectorSubcoreMesh(core_axis_name="c", subcore_axis_name="s"),
        out_type=jax.ShapeDtypeStruct((B * EMB_DIM * hw,), jnp.float32),
        scratch_types=[
            pltpu.VMEM((D_PER_W * NUM_EMB,), jnp.float32),
            pltpu.VMEM((hw,), jnp.int32),
            pltpu.VMEM((D_PER_W * HW,), jnp.float32),
        ],
        compiler_params=pltpu.CompilerParams(needs_layout_passes=False),
    )(_sc_gather_kernel)
    qflat = sc_gather(cbt, codes2d)

    quantized_x = qflat.reshape(B, D, H, W)
    latent_loss = 2.0 * jnp.sum(loss_sum) / (B * hw * D)
    return quantized_x, codes2d, latent_loss


# speedup vs baseline: 1.9844x; 1.9844x over previous
import functools

import jax
import jax.numpy as jnp
from jax.experimental import pallas as pl
from jax.experimental.pallas import tpu as pltpu

NUM_EMB = 1024
EMB_DIM = 64
BPB = 2
HW = 1024


def _vq_kernel(x_ref, cb_ref, s2_ref, iota_row_ref, iota_col_ref,
               out_ref, codes_ref, loss_ref):
    x_blk = x_ref[...]
    cb = cb_ref[...]
    flat = jnp.transpose(x_blk, (0, 2, 1)).reshape(BPB * HW, EMB_DIM)

    m2 = jax.lax.dot_general(
        flat, cb + cb,
        dimension_numbers=(((1,), (1,)), ((), ())),
        preferred_element_type=jnp.float32,
    )
    s1 = jnp.sum(flat * flat, axis=1, keepdims=True)
    s2 = s2_ref[...]
    d2 = (s1 - m2) + s2

    minv = jnp.min(d2, axis=1, keepdims=True)
    iota_row = iota_row_ref[...]
    codes_f = jnp.min(jnp.where(d2 == minv, iota_row, jnp.float32(NUM_EMB)),
                      axis=1)
    codes_ref[0, 0, :] = codes_f.astype(jnp.int32)

    iota_col = iota_col_ref[...]
    onehot = (iota_col == codes_f[None, :]).astype(jnp.bfloat16)
    q_t = jax.lax.dot_general(
        cb.astype(jnp.bfloat16), onehot,
        dimension_numbers=(((0,), (0,)), ((), ())),
        preferred_element_type=jnp.float32,
    )

    for i in range(BPB):
        out_ref[i] = q_t[:, i * HW:(i + 1) * HW]
    loss_ref[0, 0, 0] = jnp.sum(minv)


@functools.partial(jax.jit, static_argnames=())
def kernel(x, codebook):
    B, D, H, W = x.shape
    hw = H * W
    nsteps = B // BPB
    x3 = x.reshape(B, D, hw)
    s2 = jnp.sum(codebook ** 2, axis=1)[None, :]
    iota_row = jax.lax.iota(jnp.float32, NUM_EMB)[None, :]
    iota_col = jax.lax.iota(jnp.float32, NUM_EMB)[:, None]

    out, codes3, loss_sum = pl.pallas_call(
        _vq_kernel,
        grid=(nsteps,),
        in_specs=[
            pl.BlockSpec((BPB, D, hw), lambda b: (b, 0, 0)),
            pl.BlockSpec((NUM_EMB, EMB_DIM), lambda b: (0, 0)),
            pl.BlockSpec((1, NUM_EMB), lambda b: (0, 0)),
            pl.BlockSpec((1, NUM_EMB), lambda b: (0, 0)),
            pl.BlockSpec((NUM_EMB, 1), lambda b: (0, 0)),
        ],
        out_specs=[
            pl.BlockSpec((BPB, D, hw), lambda b: (b, 0, 0)),
            pl.BlockSpec((1, 1, BPB * hw), lambda b: (b, 0, 0)),
            pl.BlockSpec((1, 1, 1), lambda b: (b, 0, 0), memory_space=pltpu.SMEM),
        ],
        out_shape=[
            jax.ShapeDtypeStruct((B, D, hw), jnp.float32),
            jax.ShapeDtypeStruct((nsteps, 1, BPB * hw), jnp.int32),
            jax.ShapeDtypeStruct((nsteps, 1, 1), jnp.float32),
        ],
        compiler_params=pltpu.CompilerParams(
            dimension_semantics=("parallel",),
        ),
    )(x3, codebook, s2, iota_row, iota_col)

    quantized_x = out.reshape(B, D, H, W)
    codes = codes3.reshape(B, hw)
    latent_loss = 2.0 * jnp.sum(loss_sum) / (B * hw * D)
    return quantized_x, codes, latent_loss


# speedup vs baseline: 2.1466x; 1.0817x over previous
import functools

import jax
import jax.numpy as jnp
from jax.experimental import pallas as pl
from jax.experimental.pallas import tpu as pltpu

NUM_EMB = 1024
EMB_DIM = 64
BPB = 4
HW = 1024


def _vq_kernel(x_ref, cb_ref, s2_ref, iota_row_ref, iota_col_ref,
               out_ref, codes_ref, loss_ref):
    x_blk = x_ref[...]
    cb = cb_ref[...]
    flat = jnp.transpose(x_blk, (0, 2, 1)).reshape(BPB * HW, EMB_DIM)

    m2 = jax.lax.dot_general(
        flat, cb + cb,
        dimension_numbers=(((1,), (1,)), ((), ())),
        preferred_element_type=jnp.float32,
    )
    s1 = jnp.sum(flat * flat, axis=1, keepdims=True)
    s2 = s2_ref[...]
    d2 = (s1 - m2) + s2

    minv = jnp.min(d2, axis=1, keepdims=True)
    iota_row = iota_row_ref[...]
    codes_f = jnp.min(jnp.where(d2 == minv, iota_row, jnp.float32(NUM_EMB)),
                      axis=1)
    codes_ref[0, 0, :] = codes_f.astype(jnp.int32)

    iota_col = iota_col_ref[...]
    onehot = (iota_col == codes_f[None, :]).astype(jnp.bfloat16)
    q_t = jax.lax.dot_general(
        cb.astype(jnp.bfloat16), onehot,
        dimension_numbers=(((0,), (0,)), ((), ())),
        preferred_element_type=jnp.float32,
    )

    loss = jnp.float32(0.0)
    for i in range(BPB):
        q_i = q_t[:, i * HW:(i + 1) * HW]
        out_ref[i] = q_i
        diff = q_i - x_blk[i]
        loss += jnp.sum(diff * diff)
    loss_ref[0, 0, 0] = loss


@functools.partial(jax.jit, static_argnames=())
def kernel(x, codebook):
    B, D, H, W = x.shape
    hw = H * W
    nsteps = B // BPB
    x3 = x.reshape(B, D, hw)
    s2 = jnp.sum(codebook ** 2, axis=1)[None, :]
    iota_row = jax.lax.iota(jnp.float32, NUM_EMB)[None, :]
    iota_col = jax.lax.iota(jnp.float32, NUM_EMB)[:, None]

    out, codes3, loss_sum = pl.pallas_call(
        _vq_kernel,
        grid=(nsteps,),
        in_specs=[
            pl.BlockSpec((BPB, D, hw), lambda b: (b, 0, 0)),
            pl.BlockSpec((NUM_EMB, EMB_DIM), lambda b: (0, 0)),
            pl.BlockSpec((1, NUM_EMB), lambda b: (0, 0)),
            pl.BlockSpec((1, NUM_EMB), lambda b: (0, 0)),
            pl.BlockSpec((NUM_EMB, 1), lambda b: (0, 0)),
        ],
        out_specs=[
            pl.BlockSpec((BPB, D, hw), lambda b: (b, 0, 0)),
            pl.BlockSpec((1, 1, BPB * hw), lambda b: (b, 0, 0)),
            pl.BlockSpec((1, 1, 1), lambda b: (b, 0, 0), memory_space=pltpu.SMEM),
        ],
        out_shape=[
            jax.ShapeDtypeStruct((B, D, hw), jnp.float32),
            jax.ShapeDtypeStruct((nsteps, 1, BPB * hw), jnp.int32),
            jax.ShapeDtypeStruct((nsteps, 1, 1), jnp.float32),
        ],
        compiler_params=pltpu.CompilerParams(
            dimension_semantics=("parallel",),
        ),
    )(x3, codebook, s2, iota_row, iota_col)

    quantized_x = out.reshape(B, D, H, W)
    codes = codes3.reshape(B, hw)
    latent_loss = 2.0 * jnp.sum(loss_sum) / (B * hw * D)
    return quantized_x, codes, latent_loss
